# Initial kernel scaffold; baseline (speedup 1.0000x reference)
import jax
import jax.numpy as jnp
from jax.experimental import pallas as pl


def kernel(features, labels):
    raise NotImplementedError("write your pallas kernel here")


# speedup vs baseline: 2.0433x; 2.0433x over previous
import functools

import jax
import jax.numpy as jnp
from jax import lax
from jax.experimental import pallas as pl
from jax.experimental.pallas import tpu as pltpu
from jax.experimental.pallas import tpu_sc as plsc

N = 160000
D = 256
K = 1000
KPAD = 1024
NC = 2
NS = 16
NW = NC * NS
CPT = KPAD // NW
CHUNK = 256
LANES = 16
NJ = D // LANES
NBLK = N // 8


def _seg_call(features, labels):
    mesh = plsc.VectorSubcoreMesh(core_axis_name="c", subcore_axis_name="s",
                                  num_cores=NC, num_subcores=NS)

    @functools.partial(
        pl.kernel,
        mesh=mesh,
        out_type=(
            jax.ShapeDtypeStruct((KPAD, D), jnp.float32),
            jax.ShapeDtypeStruct((KPAD, LANES), jnp.float32),
            jax.ShapeDtypeStruct((NW, LANES), jnp.float32),
        ),
        scratch_types=[
            pltpu.VMEM((CHUNK, D), jnp.float32),
            pltpu.VMEM((CHUNK,), jnp.int32),
            pltpu.VMEM((CPT + 1, D), jnp.float32),
            pltpu.VMEM((CPT + 1, LANES), jnp.float32),
            pltpu.VMEM((16,), jnp.int32),
            pltpu.VMEM((16,), jnp.int32),
            pltpu.VMEM((LANES,), jnp.float32),
            pltpu.VMEM((2000,), jnp.int32),
            pltpu.VMEM_SHARED((N + 16,), jnp.int32),
        ],
    )
    def seg(feat_hbm, lab_hbm, sums_out, cnt_out, sx_out,
            rows_v, lab_v, acc_s, acc_c, probe_v, sent_v, sq_v, stage_v,
            lab_sp):
        cid = lax.axis_index("c")
        sid = lax.axis_index("s")
        wid = cid * NS + sid
        k_lo = wid * CPT

        seg_len = N // NS

        def stage_body(i, carry):
            off = sid * seg_len + i * 2000
            pltpu.sync_copy(lab_hbm.at[pl.ds(off, 2000)], stage_v)
            pltpu.sync_copy(stage_v, lab_sp.at[pl.ds(off, 2000)])
            return carry

        lax.fori_loop(0, seg_len // 2000, stage_body, 0)
        sent_v[...] = jnp.full((16,), jnp.int32(KPAD + 1), jnp.int32)

        @pl.when(sid == 0)
        def _():
            pltpu.sync_copy(sent_v, lab_sp.at[pl.ds(N, 16)])

        zf = jnp.zeros((LANES,), jnp.float32)

        def zbody(r, carry):
            for j in range(NJ):
                acc_s[r, pl.ds(j * LANES, LANES)] = zf
            acc_c[r] = zf
            return carry

        lax.fori_loop(0, CPT + 1, zbody, 0)
        plsc.subcore_barrier()

        def lower_bound(tgt):
            def bb(_, lh):
                lo, hi = lh
                mid = jnp.minimum((lo + hi) // 2, NBLK - 1)
                pltpu.sync_copy(lab_sp.at[pl.ds(mid * 8, 16)], probe_v)
                ge = probe_v[...][0] >= tgt
                active = lo < hi
                new_lo = jnp.where(jnp.logical_and(active, jnp.logical_not(ge)),
                                   mid + 1, lo)
                new_hi = jnp.where(jnp.logical_and(active, ge), mid, hi)
                return (new_lo, new_hi)

            lo, _ = lax.fori_loop(0, 15, bb, (jnp.int32(0), jnp.int32(NBLK)))
            blk = jnp.maximum(lo - 1, 0)
            pltpu.sync_copy(lab_sp.at[pl.ds(blk * 8, 16)], probe_v)
            pv = probe_v[...]
            cnt_lt = jnp.int32(0)
            for j in range(16):
                cnt_lt = cnt_lt + jnp.where(pv[j] < tgt, jnp.int32(1),
                                            jnp.int32(0))
            return jnp.where(lo == 0, jnp.int32(0), blk * 8 + cnt_lt)

        row_lo = lower_bound(k_lo)
        row_hi = lower_bound(k_lo + CPT)

        start0 = (row_lo // 8) * 8
        span = row_hi - start0
        nchunks = (span + CHUNK - 1) // CHUNK

        def chunk_body(t, carry):
            prev, cnt, sums, sqs = carry
            nominal = start0 + t * CHUNK
            cur = jnp.minimum(nominal, N - CHUNK)
            vstart = jnp.maximum(nominal, row_lo)
            pltpu.sync_copy(feat_hbm.at[pl.ds(cur, CHUNK)], rows_v)
            pltpu.sync_copy(lab_hbm.at[pl.ds(cur, CHUNK)], lab_v)

            def grp_body(g, rc):
                prev, cnt, sums, sqs = rc
                lv = lab_v[pl.ds(g * LANES, LANES)]
                for rr in range(LANES):
                    c = lv[rr]
                    r = g * LANES + rr
                    gidx = cur + r
                    valid = jnp.logical_and(gidx >= vstart, gidx < row_hi)
                    eff_same = jnp.logical_or(jnp.logical_not(valid), c == prev)
                    idx = jnp.where(valid, c - k_lo, jnp.int32(CPT))
                    cnt = (jnp.where(eff_same, cnt, 0.0)
                           + jnp.where(valid, 1.0, 0.0))
                    acc_c[idx] = jnp.full((LANES,), cnt, jnp.float32)
                    new_sums = []
                    new_sqs = []
                    for j in range(NJ):
                        x = rows_v[r, pl.ds(j * LANES, LANES)]
                        xx = jnp.where(valid, x, 0.0)
                        s = jnp.where(eff_same, sums[j], 0.0) + xx
                        acc_s[idx, pl.ds(j * LANES, LANES)] = s
                        new_sums.append(s)
                        new_sqs.append(sqs[j] + xx * xx)
                    sums = tuple(new_sums)
                    sqs = tuple(new_sqs)
                    prev = jnp.where(valid, c, prev)
                return (prev, cnt, sums, sqs)

            return lax.fori_loop(0, CHUNK // LANES, grp_body,
                                 (prev, cnt, sums, sqs))

        init = (jnp.int32(-1), jnp.float32(0.0), (zf,) * NJ, (zf,) * NJ)
        _, _, _, sqs = lax.fori_loop(0, nchunks, chunk_body, init)

        tot = sqs[0]
        for j in range(1, NJ):
            tot = tot + sqs[j]
        sq_v[...] = tot

        pltpu.sync_copy(acc_s.at[pl.ds(0, CPT)], sums_out.at[pl.ds(k_lo, CPT)])
        pltpu.sync_copy(acc_c.at[pl.ds(0, CPT)], cnt_out.at[pl.ds(k_lo, CPT)])
        pltpu.sync_copy(sq_v, sx_out.at[wid])

    return seg(features, labels)


def _combine_body(s_ref, c_ref, sx_ref, means_ref, comp_ref, sep_ref):
    sums = s_ref[...]
    counts = c_ref[:, 0:1]
    safe = jnp.maximum(counts, 1.0)
    means = sums / safe
    means_ref[...] = means
    msq = jnp.sum(sums * means)
    total = jnp.sum(sums, axis=0)
    sx = jnp.sum(sx_ref[...])
    comp_ref[0, 0] = sx - msq
    sep_ref[0, 0] = msq - jnp.sum(total * total) / jnp.float32(N)


def _combine(sums, cnt16, sxp):
    return pl.pallas_call(
        _combine_body,
        in_specs=[
            pl.BlockSpec(memory_space=pltpu.VMEM),
            pl.BlockSpec(memory_space=pltpu.VMEM),
            pl.BlockSpec(memory_space=pltpu.VMEM),
        ],
        out_specs=[
            pl.BlockSpec(memory_space=pltpu.VMEM),
            pl.BlockSpec(memory_space=pltpu.SMEM),
            pl.BlockSpec(memory_space=pltpu.SMEM),
        ],
        out_shape=[
            jax.ShapeDtypeStruct((KPAD, D), jnp.float32),
            jax.ShapeDtypeStruct((1, 1), jnp.float32),
            jax.ShapeDtypeStruct((1, 1), jnp.float32),
        ],
    )(sums, cnt16, sxp)


@jax.jit
def kernel(features, labels):
    labels = labels.astype(jnp.int32)
    sums, cnt16, sxp = _seg_call(features, labels)
    means, comp, sep = _combine(sums, cnt16, sxp)
    all_means = means[:K][:, None, :]
    return (comp.reshape(1), sep.reshape(1), all_means)


# speedup vs baseline: 13.4773x; 6.5959x over previous
import functools

import jax
import jax.numpy as jnp
from jax import lax
from jax.experimental import pallas as pl
from jax.experimental.pallas import tpu as pltpu
from jax.experimental.pallas import tpu_sc as plsc

N = 160000
D = 256
K = 1000
KPAD = 1024
NC = 2
NS = 16
NW = NC * NS
CPT = KPAD // NW
CHUNK = 192
LANES = 16
NJ = D // LANES
NBLK = N // 8
NWIN = CHUNK // LANES


def _seg_call(features, labels):
    mesh = plsc.VectorSubcoreMesh(core_axis_name="c", subcore_axis_name="s",
                                  num_cores=NC, num_subcores=NS)

    @functools.partial(
        pl.kernel,
        mesh=mesh,
        out_type=(
            jax.ShapeDtypeStruct((KPAD, D), jnp.float32),
            jax.ShapeDtypeStruct((KPAD, LANES), jnp.float32),
            jax.ShapeDtypeStruct((NW, LANES), jnp.float32),
        ),
        scratch_types=[
            pltpu.VMEM((CHUNK, D), jnp.float32),
            pltpu.VMEM((CHUNK, D), jnp.float32),
            pltpu.VMEM((CHUNK,), jnp.int32),
            pltpu.VMEM((CPT + 1, D), jnp.float32),
            pltpu.VMEM((CPT + 1, LANES), jnp.float32),
            pltpu.VMEM((16,), jnp.int32),
            pltpu.VMEM((16,), jnp.int32),
            pltpu.VMEM((LANES,), jnp.float32),
            pltpu.VMEM((2000,), jnp.int32),
            pltpu.VMEM_SHARED((N + 16,), jnp.int32),
            pltpu.SemaphoreType.DMA,
            pltpu.SemaphoreType.DMA,
        ],
    )
    def seg(feat_hbm, lab_hbm, sums_out, cnt_out, sx_out,
            rows_a, rows_b, lab_v, acc_s, acc_c, probe_v, sent_v, sq_v,
            stage_v, lab_sp, sem_a, sem_b):
        cid = lax.axis_index("c")
        sid = lax.axis_index("s")
        wid = cid * NS + sid
        k_lo = wid * CPT

        seg_len = N // NS

        def stage_body(i, carry):
            off = sid * seg_len + i * 2000
            pltpu.sync_copy(lab_hbm.at[pl.ds(off, 2000)], stage_v)
            pltpu.sync_copy(stage_v, lab_sp.at[pl.ds(off, 2000)])
            return carry

        lax.fori_loop(0, seg_len // 2000, stage_body, 0)
        sent_v[...] = jnp.full((16,), jnp.int32(KPAD + 1), jnp.int32)

        @pl.when(sid == 0)
        def _():
            pltpu.sync_copy(sent_v, lab_sp.at[pl.ds(N, 16)])

        zf = jnp.zeros((LANES,), jnp.float32)

        def zbody(r, carry):
            for j in range(NJ):
                acc_s[r, pl.ds(j * LANES, LANES)] = zf
            acc_c[r] = zf
            return carry

        lax.fori_loop(0, CPT + 1, zbody, 0)
        plsc.subcore_barrier()

        def lower_bound(tgt):
            def bb(_, lh):
                lo, hi = lh
                mid = jnp.minimum((lo + hi) // 2, NBLK - 1)
                pltpu.sync_copy(lab_sp.at[pl.ds(mid * 8, 16)], probe_v)
                ge = probe_v[...][0] >= tgt
                active = lo < hi
                new_lo = jnp.where(jnp.logical_and(active, jnp.logical_not(ge)),
                                   mid + 1, lo)
                new_hi = jnp.where(jnp.logical_and(active, ge), mid, hi)
                return (new_lo, new_hi)

            lo, _ = lax.fori_loop(0, 15, bb, (jnp.int32(0), jnp.int32(NBLK)))
            blk = jnp.maximum(lo - 1, 0)
            pltpu.sync_copy(lab_sp.at[pl.ds(blk * 8, 16)], probe_v)
            pv = probe_v[...]
            cnt_lt = jnp.int32(0)
            for j in range(16):
                cnt_lt = cnt_lt + jnp.where(pv[j] < tgt, jnp.int32(1),
                                            jnp.int32(0))
            return jnp.where(lo == 0, jnp.int32(0), blk * 8 + cnt_lt)

        row_lo = lower_bound(k_lo)
        row_hi = lower_bound(k_lo + CPT)

        start0 = (row_lo // 8) * 8
        span = row_hi - start0
        nchunks = (span + CHUNK - 1) // CHUNK

        def cur_of(t):
            return jnp.minimum(start0 + t * CHUNK, N - CHUNK)

        def process(t, buf, sqs):
            nominal = start0 + t * CHUNK
            cur = cur_of(t)
            vstart = jnp.maximum(nominal, row_lo)
            pltpu.sync_copy(lab_sp.at[pl.ds(cur, CHUNK)], lab_v)
            lv0 = lab_v[pl.ds(0, LANES)][0]
            lvl = lab_v[pl.ds(CHUNK - LANES, LANES)][LANES - 1]
            c_begin = jnp.maximum(lv0, k_lo)
            c_end = jnp.where(t < nchunks,
                              jnp.maximum(jnp.minimum(lvl + 1, k_lo + CPT),
                                          c_begin),
                              c_begin)

            def clb(tgt):
                def bb(_, lh):
                    lo, hi = lh
                    mid = jnp.minimum((lo + hi) // 2, NWIN - 1)
                    head = lab_v[pl.ds(mid * LANES, LANES)][0]
                    ge = head >= tgt
                    active = lo < hi
                    new_lo = jnp.where(
                        jnp.logical_and(active, jnp.logical_not(ge)),
                        mid + 1, lo)
                    new_hi = jnp.where(jnp.logical_and(active, ge), mid, hi)
                    return (new_lo, new_hi)

                lo, _ = lax.fori_loop(0, 4, bb,
                                      (jnp.int32(0), jnp.int32(NWIN)))
                blk = jnp.maximum(lo - 1, 0)
                pv = lab_v[pl.ds(blk * LANES, LANES)]
                cnt = jnp.int32(0)
                for j in range(LANES):
                    cnt = cnt + jnp.where(pv[j] < tgt, jnp.int32(1),
                                          jnp.int32(0))
                return jnp.where(lo == 0, jnp.int32(0), blk * LANES + cnt)

            def class_body(c, carry):
                prev_hi, sqs = carry
                hi_g = clb(c + 1)
                ci = c - k_lo
                lo_r = jnp.maximum(prev_hi, vstart - cur)
                hi_r = jnp.maximum(hi_g, lo_r)
                accs = tuple(acc_s[ci, pl.ds(j * LANES, LANES)]
                             for j in range(NJ))

                def row_body(r, rc):
                    accs, sqs = rc
                    na = []
                    nq = []
                    for j in range(NJ):
                        x = buf[r, pl.ds(j * LANES, LANES)]
                        na.append(accs[j] + x)
                        nq.append(sqs[j] + x * x)
                    return (tuple(na), tuple(nq))

                accs, sqs = lax.fori_loop(lo_r, hi_r, row_body, (accs, sqs))
                for j in range(NJ):
                    acc_s[ci, pl.ds(j * LANES, LANES)] = accs[j]
                nrows = (hi_r - lo_r).astype(jnp.float32)
                acc_c[ci] = acc_c[ci] + jnp.full((LANES,), 1.0,
                                                 jnp.float32) * nrows
                return (hi_g, sqs)

            init_hi = clb(c_begin)
            _, sqs = lax.fori_loop(c_begin, c_end, class_body,
                                   (init_hi, sqs))
            return sqs

        @pl.when(nchunks > 0)
        def _():
            pltpu.async_copy(feat_hbm.at[pl.ds(cur_of(0), CHUNK)],
                             rows_a, sem_a)

        npairs = (nchunks + 1) // 2
        nceil = 2 * npairs

        def pair_body(p, sqs):
            t0 = 2 * p
            pltpu.async_copy(feat_hbm.at[pl.ds(cur_of(t0 + 1), CHUNK)],
                             rows_b, sem_b)
            pltpu.make_async_copy(feat_hbm.at[pl.ds(cur_of(t0), CHUNK)],
                                  rows_a, sem_a).wait()
            sqs = process(t0, rows_a, sqs)

            @pl.when(t0 + 2 < nceil)
            def _():
                pltpu.async_copy(feat_hbm.at[pl.ds(cur_of(t0 + 2), CHUNK)],
                                 rows_a, sem_a)

            pltpu.make_async_copy(feat_hbm.at[pl.ds(cur_of(t0 + 1), CHUNK)],
                                  rows_b, sem_b).wait()
            sqs = process(t0 + 1, rows_b, sqs)
            return sqs

        sqs = lax.fori_loop(0, npairs, pair_body, (zf,) * NJ)

        tot = sqs[0]
        for j in range(1, NJ):
            tot = tot + sqs[j]
        sq_v[...] = tot

        pltpu.sync_copy(acc_s.at[pl.ds(0, CPT)], sums_out.at[pl.ds(k_lo, CPT)])
        pltpu.sync_copy(acc_c.at[pl.ds(0, CPT)], cnt_out.at[pl.ds(k_lo, CPT)])
        pltpu.sync_copy(sq_v, sx_out.at[wid])

    return seg(features, labels)


def _combine_body(s_ref, c_ref, sx_ref, means_ref, comp_ref, sep_ref):
    sums = s_ref[...]
    counts = c_ref[:, 0:1]
    safe = jnp.maximum(counts, 1.0)
    means = sums / safe
    means_ref[...] = means
    msq = jnp.sum(sums * means)
    total = jnp.sum(sums, axis=0)
    sx = jnp.sum(sx_ref[...])
    comp_ref[0, 0] = sx - msq
    sep_ref[0, 0] = msq - jnp.sum(total * total) / jnp.float32(N)


def _combine(sums, cnt16, sxp):
    return pl.pallas_call(
        _combine_body,
        in_specs=[
            pl.BlockSpec(memory_space=pltpu.VMEM),
            pl.BlockSpec(memory_space=pltpu.VMEM),
            pl.BlockSpec(memory_space=pltpu.VMEM),
        ],
        out_specs=[
            pl.BlockSpec(memory_space=pltpu.VMEM),
            pl.BlockSpec(memory_space=pltpu.SMEM),
            pl.BlockSpec(memory_space=pltpu.SMEM),
        ],
        out_shape=[
            jax.ShapeDtypeStruct((KPAD, D), jnp.float32),
            jax.ShapeDtypeStruct((1, 1), jnp.float32),
            jax.ShapeDtypeStruct((1, 1), jnp.float32),
        ],
    )(sums, cnt16, sxp)


@jax.jit
def kernel(features, labels):
    labels = labels.astype(jnp.int32)
    sums, cnt16, sxp = _seg_call(features, labels)
    means, comp, sep = _combine(sums, cnt16, sxp)
    all_means = means[:K][:, None, :]
    return (comp.reshape(1), sep.reshape(1), all_means)
